# baseline (speedup 1.0000x reference)
import functools

import jax
import jax.numpy as jnp
from jax import lax
from jax.experimental import pallas as pl
from jax.experimental.pallas import tpu as pltpu
from jax.experimental.pallas import tpu_sc as plsc

_VOCAB = 1000000
_DIM = 32
_BATCH = 16384

_info = plsc.get_sparse_core_info()
_NC, _NS = _info.num_cores, _info.num_subcores
_NW = _NC * _NS
_BPW = _BATCH // _NW
_CHUNK = 128
_NCHUNK = _BPW // _CHUNK


def _make_emb():
    mesh = plsc.VectorSubcoreMesh(core_axis_name="c", subcore_axis_name="s")

    @functools.partial(
        pl.kernel,
        mesh=mesh,
        out_type=jax.ShapeDtypeStruct((_BATCH, _DIM), jnp.float32),
        scratch_types=[
            pltpu.VMEM((_NCHUNK, _CHUNK), jnp.int32),
            pltpu.VMEM((_BPW, _DIM), jnp.float32),
            pltpu.SemaphoreType.DMA,
        ],
        compiler_params=pltpu.CompilerParams(use_tc_tiling_on_sc=False),
    )
    def emb(table_hbm, idx_hbm, out_hbm, idx_v, rows_v, sem):
        wid = lax.axis_index("s") * _NC + lax.axis_index("c")
        pltpu.sync_copy(idx_hbm.at[pl.ds(wid * _NCHUNK, _NCHUNK)], idx_v)
        copies = []
        for j in range(_NCHUNK):
            copies.append(
                pltpu.async_copy(
                    table_hbm.at[idx_v.at[j]],
                    rows_v.at[pl.ds(j * _CHUNK, _CHUNK)],
                    sem,
                )
            )
        for c in copies:
            c.wait()
        pltpu.sync_copy(rows_v, out_hbm.at[pl.ds(wid * _BPW, _BPW)])

    return emb


_emb = _make_emb()


def kernel(y, table):
    idx = y.astype(jnp.int32).reshape(_NW * _NCHUNK, _CHUNK)
    return _emb(table, idx)


# speedup vs baseline: 1.3663x; 1.3663x over previous
import functools

import jax
import jax.numpy as jnp
from jax import lax
from jax.experimental import pallas as pl
from jax.experimental.pallas import tpu as pltpu
from jax.experimental.pallas import tpu_sc as plsc

_VOCAB = 1000000
_DIM = 32
_BATCH = 16384

_info = plsc.get_sparse_core_info()
_NC, _NS = _info.num_cores, _info.num_subcores
_NW = _NC * _NS
_BPW = _BATCH // _NW
_WIN = 512
_NFULL = _VOCAB // _WIN
_TAILW = _NFULL
_TAIL = _VOCAB - _NFULL * _WIN
_NCHUNK = _BATCH // 16
_SUBCAP = _BATCH


def _extract_and_scatter(w, nsub, lcol, lrank, win_v, sub_c, sub_r,
                         rows_stage, ridx_v, scr_hbm, sem, nlist):
    iota = lax.iota(jnp.int32, 16)

    def scan(i, ns):
        cols = lcol[pl.ds(i * 16, 16)]
        rks = lrank[pl.ds(i * 16, 16)]
        valid = (i * 16 + iota) < nlist
        m = jnp.logical_and((cols >> 9) == w, valid)
        plsc.store_compressed(sub_c.at[pl.ds(ns, 16)], cols, mask=m)
        plsc.store_compressed(sub_r.at[pl.ds(ns, 16)], rks, mask=m)
        return ns + plsc.all_reduce_population_count(m)[0]

    nsub = lax.fori_loop(0, (nlist + 15) // 16, scan, 0)

    def batch(b, _):
        for g in range(8):
            gbase = b * 128 + g * 16
            cols16 = sub_c[pl.ds(gbase, 16)] & (_WIN - 1)
            rk16 = sub_r[pl.ds(gbase, 16)]
            pos16 = g * 16 + iota
            for d in range(_DIM):
                dfull = jnp.full((16,), d, jnp.int32)
                vals = plsc.load_gather(win_v, [dfull, cols16])
                plsc.store_scatter(rows_stage, [pos16, dfull], vals)
            valid = (gbase + iota) < nsub
            ridx_v[0, pl.ds(g * 16, 16)] = jnp.where(valid, rk16, -1)
        pltpu.async_copy(
            rows_stage,
            scr_hbm.at[plsc.Indices(ridx_v.at[0], ignored_value=-1)],
            sem,
        ).wait()
        return _

    lax.fori_loop(0, (nsub + 127) // 128, batch, 0)


def _make_k1():
    mesh = plsc.VectorSubcoreMesh(core_axis_name="c", subcore_axis_name="s")

    @functools.partial(
        pl.kernel,
        mesh=mesh,
        out_type=jax.ShapeDtypeStruct((_BATCH, 128), jnp.float32),
        scratch_types=[
            pltpu.VMEM((_BATCH,), jnp.int32),
            pltpu.VMEM((_BATCH,), jnp.int32),
            pltpu.VMEM((_BATCH,), jnp.int32),
            pltpu.VMEM((_DIM, _WIN), jnp.float32),
            pltpu.VMEM((_SUBCAP,), jnp.int32),
            pltpu.VMEM((_SUBCAP,), jnp.int32),
            pltpu.VMEM((128, 128), jnp.float32),
            pltpu.VMEM((1, 128), jnp.int32),
            pltpu.SemaphoreType.DMA,
        ],
        compiler_params=pltpu.CompilerParams(needs_layout_passes=False),
    )
    def k1(tb_hbm, idx_hbm, tail_hbm, scr_hbm, y_v, lcol, lrank, win_v,
           sub_c, sub_r, rows_stage, ridx_v, sem):
        me = lax.axis_index("s") * _NC + lax.axis_index("c")
        iota = lax.iota(jnp.int32, 16)
        pltpu.sync_copy(idx_hbm, y_v)

        def build(i, cnt):
            ys = y_v[pl.ds(i * 16, 16)]
            m = ((ys >> 9) & (_NW - 1)) == me
            plsc.store_compressed(lcol.at[pl.ds(cnt, 16)], ys, mask=m)
            plsc.store_compressed(
                lrank.at[pl.ds(cnt, 16)], i * 16 + iota, mask=m
            )
            return cnt + plsc.all_reduce_population_count(m)[0]

        nlist = lax.fori_loop(0, _NCHUNK, build, 0)

        def window(k, _):
            w = me + k * _NW

            @pl.when(w < _NFULL)
            def _go():
                pltpu.sync_copy(tb_hbm.at[:, pl.ds(w * _WIN, _WIN)], win_v)
                _extract_and_scatter(w, 0, lcol, lrank, win_v, sub_c, sub_r,
                                     rows_stage, ridx_v, scr_hbm, sem, nlist)

            return _

        lax.fori_loop(0, (_NFULL + _NW - 1) // _NW, window, 0)

        @pl.when(me == _TAILW % _NW)
        def _tail():
            pltpu.sync_copy(tail_hbm, win_v)
            _extract_and_scatter(_TAILW, 0, lcol, lrank, win_v, sub_c, sub_r,
                                 rows_stage, ridx_v, scr_hbm, sem, nlist)

    return k1


def _make_k2():
    mesh = plsc.VectorSubcoreMesh(core_axis_name="c", subcore_axis_name="s")

    @functools.partial(
        pl.kernel,
        mesh=mesh,
        out_type=jax.ShapeDtypeStruct((_DIM, _BATCH), jnp.float32),
        scratch_types=[
            pltpu.VMEM((_BPW, 128), jnp.float32),
            pltpu.VMEM((_DIM, _BPW), jnp.float32),
        ],
        compiler_params=pltpu.CompilerParams(needs_layout_passes=False),
    )
    def k2(scr_hbm, out_hbm, rows2_v, cols_v):
        me = lax.axis_index("s") * _NC + lax.axis_index("c")
        iota = lax.iota(jnp.int32, 16)
        base = me * _BPW
        pltpu.sync_copy(scr_hbm.at[pl.ds(base, _BPW)], rows2_v)

        def transpose_d(d, _):
            dfull = jnp.full((16,), d, jnp.int32)
            for g in range(_BPW // 16):
                vals = plsc.load_gather(rows2_v, [g * 16 + iota, dfull])
                cols_v[d, pl.ds(g * 16, 16)] = vals
            return _

        lax.fori_loop(0, _DIM, transpose_d, 0)
        pltpu.sync_copy(cols_v, out_hbm.at[:, pl.ds(base, _BPW)])

    return k2


_k1 = _make_k1()
_k2 = _make_k2()


def kernel(y, table):
    idx = y.astype(jnp.int32)
    tail_win = jnp.pad(
        table[_NFULL * _WIN :].T, ((0, 0), (0, _WIN - _TAIL))
    )
    scr = _k1(table.T, idx, tail_win)
    out_t = _k2(scr)
    return out_t.T


# speedup vs baseline: 4.0385x; 2.9559x over previous
import functools

import jax
import jax.numpy as jnp
from jax import lax
from jax.experimental import pallas as pl
from jax.experimental.pallas import tpu as pltpu
from jax.experimental.pallas import tpu_sc as plsc

_VOCAB = 1000000
_DIM = 32
_BATCH = 16384

_info = plsc.get_sparse_core_info()
_NC, _NS = _info.num_cores, _info.num_subcores
_NW = _NC * _NS
_BPW = _BATCH // _NW
_WIN = 512
_WSHIFT = 9
_NFULL = _VOCAB // _WIN
_TAILW = _NFULL
_TAIL = _VOCAB - _NFULL * _WIN
_NPAIR = (_NFULL + 2 * _NW - 1) // (2 * _NW)
_YCHUNK = 2048


def _process_window(w, win_ref, nlist, lcol, lrank, sub_c, sub_r,
                    rows_stage, ridx_v, flag_sm, scr_hbm, sem_s):
    iota = lax.iota(jnp.int32, 16)

    def scan(i, ns):
        cols = lcol[pl.ds(i * 16, 16)]
        rks = lrank[pl.ds(i * 16, 16)]
        valid = (i * 16 + iota) < nlist
        m = jnp.logical_and((cols >> _WSHIFT) == w, valid)
        plsc.store_compressed(sub_c.at[pl.ds(ns, 16)], cols, mask=m)
        plsc.store_compressed(sub_r.at[pl.ds(ns, 16)], rks, mask=m)
        return ns + plsc.all_reduce_population_count(m)[0]

    nsub = lax.fori_loop(0, (nlist + 15) // 16, scan, 0)
    ngrp = (nsub + 15) // 16

    def scatter_copy():
        return pltpu.make_async_copy(
            rows_stage,
            scr_hbm.at[plsc.Indices(ridx_v.at[0], ignored_value=-1)],
            sem_s,
        )

    def drain_scatter():
        @pl.when(flag_sm[0] == 1)
        def _():
            scatter_copy().wait()
            flag_sm[0] = 0

    def gloop(g, _):
        gslot = g & 7

        @pl.when(gslot == 0)
        def _fresh_batch():
            drain_scatter()
            neg1 = jnp.full((16,), -1, jnp.int32)
            for h in range(8):
                ridx_v[0, pl.ds(h * 16, 16)] = neg1

        gb = g * 16
        cols16 = sub_c[pl.ds(gb, 16)] & (_WIN - 1)
        rk16 = sub_r[pl.ds(gb, 16)]
        valid = (gb + iota) < nsub
        pos16 = gslot * 16 + iota
        for d in range(_DIM):
            dfull = jnp.full((16,), d, jnp.int32)
            vals = plsc.load_gather(win_ref, [dfull, cols16])
            plsc.store_scatter(rows_stage, [pos16, dfull], vals)
        ridx_v[0, pl.ds(gslot * 16, 16)] = jnp.where(valid, rk16, -1)

        @pl.when(gslot == 7)
        def _flush():
            scatter_copy().start()
            flag_sm[0] = 1

        return _

    lax.fori_loop(0, ngrp, gloop, 0)

    @pl.when(ngrp % 8 != 0)
    def _flush_tail():
        scatter_copy().start()
        flag_sm[0] = 1


def _make_k1():
    mesh = plsc.VectorSubcoreMesh(core_axis_name="c", subcore_axis_name="s")

    @functools.partial(
        pl.kernel,
        mesh=mesh,
        out_type=jax.ShapeDtypeStruct((_BATCH, 128), jnp.float32),
        scratch_types=[
            pltpu.VMEM((_YCHUNK,), jnp.int32),
            pltpu.VMEM((_BATCH,), jnp.int32),
            pltpu.VMEM((_BATCH,), jnp.int32),
            pltpu.VMEM((2, _DIM, _WIN), jnp.float32),
            pltpu.VMEM((_BATCH,), jnp.int32),
            pltpu.VMEM((_BATCH,), jnp.int32),
            pltpu.VMEM((128, 128), jnp.float32),
            pltpu.VMEM((1, 128), jnp.int32),
            pltpu.SMEM((8,), jnp.int32),
            pltpu.SemaphoreType.DMA,
            pltpu.SemaphoreType.DMA,
            pltpu.SemaphoreType.DMA,
        ],
        compiler_params=pltpu.CompilerParams(needs_layout_passes=False),
    )
    def k1(tb_hbm, idx_hbm, tail_hbm, scr_hbm, ybuf, lcol, lrank, win2,
           sub_c, sub_r, rows_stage, ridx_v, flag_sm, semA, semB, sem_s):
        me = lax.axis_index("s") * _NC + lax.axis_index("c")
        iota = lax.iota(jnp.int32, 16)
        flag_sm[0] = 0

        cnt = 0
        for p in range(_BATCH // _YCHUNK):
            pltpu.sync_copy(idx_hbm.at[pl.ds(p * _YCHUNK, _YCHUNK)], ybuf)

            def build(i, c, _p=p):
                ys = ybuf[pl.ds(i * 16, 16)]
                m = ((ys >> _WSHIFT) & (_NW - 1)) == me
                plsc.store_compressed(lcol.at[pl.ds(c, 16)], ys, mask=m)
                plsc.store_compressed(
                    lrank.at[pl.ds(c, 16)], _p * _YCHUNK + i * 16 + iota,
                    mask=m,
                )
                return c + plsc.all_reduce_population_count(m)[0]

            cnt = lax.fori_loop(0, _YCHUNK // 16, build, cnt)
        nlist = cnt

        def stage(w, buf, sem):
            pltpu.make_async_copy(
                tb_hbm.at[:, pl.ds(w * _WIN, _WIN)], win2.at[buf], sem
            ).start()

        def stage_wait(buf, sem):
            pltpu.make_async_copy(
                tb_hbm.at[:, pl.ds(0, _WIN)], win2.at[buf], sem
            ).wait()

        def proc(w, buf):
            _process_window(w, win2.at[buf], nlist, lcol, lrank, sub_c,
                            sub_r, rows_stage, ridx_v, flag_sm, scr_hbm,
                            sem_s)

        stage(me, 0, semA)

        def pair(pi, _):
            w0 = me + (2 * pi) * _NW
            w1 = me + (2 * pi + 1) * _NW
            w2 = me + (2 * pi + 2) * _NW

            @pl.when(w1 < _NFULL)
            def _s1():
                stage(w1, 1, semB)

            @pl.when(w0 < _NFULL)
            def _p0():
                stage_wait(0, semA)
                proc(w0, 0)

            @pl.when(w2 < _NFULL)
            def _s2():
                stage(w2, 0, semA)

            @pl.when(w1 < _NFULL)
            def _p1():
                stage_wait(1, semB)
                proc(w1, 1)

            return _

        lax.fori_loop(0, _NPAIR, pair, 0)

        @pl.when(me == _TAILW % _NW)
        def _tail():
            pltpu.sync_copy(tail_hbm, win2.at[0])
            proc(_TAILW, 0)

        @pl.when(flag_sm[0] == 1)
        def _drain():
            pltpu.make_async_copy(
                rows_stage,
                scr_hbm.at[plsc.Indices(ridx_v.at[0], ignored_value=-1)],
                sem_s,
            ).wait()
            flag_sm[0] = 0

    return k1


def _make_k2():
    mesh = plsc.VectorSubcoreMesh(core_axis_name="c", subcore_axis_name="s")

    @functools.partial(
        pl.kernel,
        mesh=mesh,
        out_type=jax.ShapeDtypeStruct((_DIM, _BATCH), jnp.float32),
        scratch_types=[
            pltpu.VMEM((_BPW, 128), jnp.float32),
            pltpu.VMEM((_DIM, _BPW), jnp.float32),
        ],
        compiler_params=pltpu.CompilerParams(needs_layout_passes=False),
    )
    def k2(scr_hbm, out_hbm, rows2_v, cols_v):
        me = lax.axis_index("s") * _NC + lax.axis_index("c")
        iota = lax.iota(jnp.int32, 16)
        base = me * _BPW
        pltpu.sync_copy(scr_hbm.at[pl.ds(base, _BPW)], rows2_v)

        def transpose_d(d, _):
            dfull = jnp.full((16,), d, jnp.int32)
            for g in range(_BPW // 16):
                vals = plsc.load_gather(rows2_v, [g * 16 + iota, dfull])
                cols_v[d, pl.ds(g * 16, 16)] = vals
            return _

        lax.fori_loop(0, _DIM, transpose_d, 0)
        pltpu.sync_copy(cols_v, out_hbm.at[:, pl.ds(base, _BPW)])

    return k2


_k1 = _make_k1()
_k2 = _make_k2()


def kernel(y, table):
    idx = y.astype(jnp.int32)
    tail_win = jnp.pad(
        table[_NFULL * _WIN :].T, ((0, 0), (0, _WIN - _TAIL))
    )
    scr = _k1(table.T, idx, tail_win)
    out_t = _k2(scr)
    return out_t.T


# speedup vs baseline: 4.2267x; 1.0466x over previous
import functools

import jax
import jax.numpy as jnp
from jax import lax
from jax.experimental import pallas as pl
from jax.experimental.pallas import tpu as pltpu
from jax.experimental.pallas import tpu_sc as plsc

_VOCAB = 1000000
_DIM = 32
_BATCH = 16384

_info = plsc.get_sparse_core_info()
_NC, _NS = _info.num_cores, _info.num_subcores
_NW = _NC * _NS
_BPW = _BATCH // _NW
_WIN = 512
_WSHIFT = 9
_NFULL = _VOCAB // _WIN
_TAILW = _NFULL
_TAIL = _VOCAB - _NFULL * _WIN
_NPAIR = (_NFULL + 2 * _NW - 1) // (2 * _NW)
_YCHUNK = 2048


def _process_window(w, win_ref, gbase, glen, srt_c, srt_r, ext_c, ext_r,
                    rows_stage, ridx_v, flag_sm, scr_hbm, sem_s):
    iota = lax.iota(jnp.int32, 16)

    def scan(i, ns):
        cols = srt_c[pl.ds(gbase + i * 16, 16)]
        rks = srt_r[pl.ds(gbase + i * 16, 16)]
        valid = (i * 16 + iota) < glen
        m = jnp.logical_and((cols >> _WSHIFT) == w, valid)
        plsc.store_compressed(ext_c.at[pl.ds(ns, 16)], cols, mask=m)
        plsc.store_compressed(ext_r.at[pl.ds(ns, 16)], rks, mask=m)
        return ns + plsc.all_reduce_population_count(m)[0]

    nsub = lax.fori_loop(0, (glen + 15) // 16, scan, 0)
    ngrp = (nsub + 15) // 16

    def scatter_copy():
        return pltpu.make_async_copy(
            rows_stage,
            scr_hbm.at[plsc.Indices(ridx_v.at[0], ignored_value=-1)],
            sem_s,
        )

    def drain_scatter():
        @pl.when(flag_sm[0] == 1)
        def _():
            scatter_copy().wait()
            flag_sm[0] = 0

    def gloop(g, _):
        gslot = g & 7

        @pl.when(gslot == 0)
        def _fresh_batch():
            drain_scatter()
            neg1 = jnp.full((16,), -1, jnp.int32)
            for h in range(8):
                ridx_v[0, pl.ds(h * 16, 16)] = neg1

        gb = g * 16
        cols16 = ext_c[pl.ds(gb, 16)] & (_WIN - 1)
        rk16 = ext_r[pl.ds(gb, 16)]
        valid = (gb + iota) < nsub
        pos16 = gslot * 16 + iota
        for d in range(_DIM):
            dfull = jnp.full((16,), d, jnp.int32)
            vals = plsc.load_gather(win_ref, [dfull, cols16])
            plsc.store_scatter(rows_stage, [pos16, dfull], vals)
        ridx_v[0, pl.ds(gslot * 16, 16)] = jnp.where(valid, rk16, -1)

        @pl.when(gslot == 7)
        def _flush():
            scatter_copy().start()
            flag_sm[0] = 1

        return _

    lax.fori_loop(0, ngrp, gloop, 0)

    @pl.when(ngrp % 8 != 0)
    def _flush_tail():
        scatter_copy().start()
        flag_sm[0] = 1


def _make_k1():
    mesh = plsc.VectorSubcoreMesh(core_axis_name="c", subcore_axis_name="s")

    @functools.partial(
        pl.kernel,
        mesh=mesh,
        out_type=jax.ShapeDtypeStruct((_BATCH, 128), jnp.float32),
        scratch_types=[
            pltpu.VMEM((_YCHUNK,), jnp.int32),
            pltpu.VMEM((_BATCH,), jnp.int32),
            pltpu.VMEM((_BATCH,), jnp.int32),
            pltpu.VMEM((2, _DIM, _WIN), jnp.float32),
            pltpu.VMEM((_BATCH,), jnp.int32),
            pltpu.VMEM((_BATCH,), jnp.int32),
            pltpu.VMEM((128, 128), jnp.float32),
            pltpu.VMEM((1, 128), jnp.int32),
            pltpu.SMEM((32,), jnp.int32),
            pltpu.SemaphoreType.DMA,
            pltpu.SemaphoreType.DMA,
            pltpu.SemaphoreType.DMA,
        ],
        compiler_params=pltpu.CompilerParams(needs_layout_passes=False),
    )
    def k1(tb_hbm, idx_hbm, tail_hbm, scr_hbm, ybuf, lcol, lrank, win2,
           sub_c, sub_r, rows_stage, ridx_v, flag_sm, semA, semB, sem_s):
        me = lax.axis_index("s") * _NC + lax.axis_index("c")
        iota = lax.iota(jnp.int32, 16)
        flag_sm[0] = 0

        cnt = 0
        for p in range(_BATCH // _YCHUNK):
            pltpu.sync_copy(idx_hbm.at[pl.ds(p * _YCHUNK, _YCHUNK)], ybuf)

            def build(i, c, _p=p):
                ys = ybuf[pl.ds(i * 16, 16)]
                m = ((ys >> _WSHIFT) & (_NW - 1)) == me
                plsc.store_compressed(lcol.at[pl.ds(c, 16)], ys, mask=m)
                plsc.store_compressed(
                    lrank.at[pl.ds(c, 16)], _p * _YCHUNK + i * 16 + iota,
                    mask=m,
                )
                return c + plsc.all_reduce_population_count(m)[0]

            cnt = lax.fori_loop(0, _YCHUNK // 16, build, cnt)
        nlist = cnt
        nchunk = (nlist + 15) // 16

        for j in range(8):
            flag_sm[8 + j] = 0

        def count_pass(i, _):
            cols = lcol[pl.ds(i * 16, 16)]
            valid = (i * 16 + iota) < nlist
            for j in range(8):
                m = jnp.logical_and((cols >> 17) == j, valid)
                flag_sm[8 + j] = (
                    flag_sm[8 + j] + plsc.all_reduce_population_count(m)[0]
                )
            return _

        lax.fori_loop(0, nchunk, count_pass, 0)
        flag_sm[16] = 0
        flag_sm[24] = 0
        for j in range(1, 8):
            flag_sm[16 + j] = flag_sm[16 + j - 1] + flag_sm[8 + j - 1]
            flag_sm[24 + j] = flag_sm[16 + j]

        def place_pass(i, _):
            cols = lcol[pl.ds(i * 16, 16)]
            rks = lrank[pl.ds(i * 16, 16)]
            valid = (i * 16 + iota) < nlist
            for j in range(8):
                m = jnp.logical_and((cols >> 17) == j, valid)
                o = flag_sm[16 + j]
                plsc.store_compressed(sub_c.at[pl.ds(o, 16)], cols, mask=m)
                plsc.store_compressed(sub_r.at[pl.ds(o, 16)], rks, mask=m)
                flag_sm[16 + j] = o + plsc.all_reduce_population_count(m)[0]
            return _

        lax.fori_loop(0, nchunk, place_pass, 0)

        def stage(w, buf, sem):
            pltpu.make_async_copy(
                tb_hbm.at[:, pl.ds(w * _WIN, _WIN)], win2.at[buf], sem
            ).start()

        def stage_wait(buf, sem):
            pltpu.make_async_copy(
                tb_hbm.at[:, pl.ds(0, _WIN)], win2.at[buf], sem
            ).wait()

        def proc(w, buf):
            j = (w - me) >> 8
            _process_window(w, win2.at[buf], flag_sm[24 + j], flag_sm[8 + j],
                            sub_c, sub_r, lcol, lrank, rows_stage, ridx_v,
                            flag_sm, scr_hbm, sem_s)

        stage(me, 0, semA)

        def pair(pi, _):
            w0 = me + (2 * pi) * _NW
            w1 = me + (2 * pi + 1) * _NW
            w2 = me + (2 * pi + 2) * _NW

            @pl.when(w1 < _NFULL)
            def _s1():
                stage(w1, 1, semB)

            @pl.when(w0 < _NFULL)
            def _p0():
                stage_wait(0, semA)
                proc(w0, 0)

            @pl.when(w2 < _NFULL)
            def _s2():
                stage(w2, 0, semA)

            @pl.when(w1 < _NFULL)
            def _p1():
                stage_wait(1, semB)
                proc(w1, 1)

            return _

        lax.fori_loop(0, _NPAIR, pair, 0)

        @pl.when(me == _TAILW % _NW)
        def _tail():
            pltpu.sync_copy(tail_hbm, win2.at[0])
            proc(_TAILW, 0)

        @pl.when(flag_sm[0] == 1)
        def _drain():
            pltpu.make_async_copy(
                rows_stage,
                scr_hbm.at[plsc.Indices(ridx_v.at[0], ignored_value=-1)],
                sem_s,
            ).wait()
            flag_sm[0] = 0

    return k1


def _make_k2():
    mesh = plsc.VectorSubcoreMesh(core_axis_name="c", subcore_axis_name="s")

    @functools.partial(
        pl.kernel,
        mesh=mesh,
        out_type=jax.ShapeDtypeStruct((_DIM, _BATCH), jnp.float32),
        scratch_types=[
            pltpu.VMEM((2, 128, 128), jnp.float32),
            pltpu.VMEM((_DIM, _BPW), jnp.float32),
            pltpu.SemaphoreType.DMA,
            pltpu.SemaphoreType.DMA,
        ],
        compiler_params=pltpu.CompilerParams(needs_layout_passes=False),
    )
    def k2(scr_hbm, out_hbm, rows2_v, cols_v, semA, semB):
        me = lax.axis_index("s") * _NC + lax.axis_index("c")
        iota = lax.iota(jnp.int32, 16)
        base = me * _BPW
        sems = (semA, semB)
        nchunks = _BPW // 128

        def chunk_copy(c, buf):
            return pltpu.make_async_copy(
                scr_hbm.at[pl.ds(base + c * 128, 128)],
                rows2_v.at[buf],
                sems[buf],
            )

        chunk_copy(0, 0).start()
        for c in range(nchunks):
            if c + 1 < nchunks:
                chunk_copy(c + 1, (c + 1) % 2).start()
            chunk_copy(c, c % 2).wait()

            def transpose_d(d, _, _c=c):
                dfull = jnp.full((16,), d, jnp.int32)
                for g in range(8):
                    vals = plsc.load_gather(
                        rows2_v.at[_c % 2], [g * 16 + iota, dfull]
                    )
                    cols_v[d, pl.ds(_c * 128 + g * 16, 16)] = vals
                return _

            lax.fori_loop(0, _DIM, transpose_d, 0)
        pltpu.sync_copy(cols_v, out_hbm.at[:, pl.ds(base, _BPW)])

    return k2


_k1 = _make_k1()
_k2 = _make_k2()


def kernel(y, table):
    idx = y.astype(jnp.int32)
    tail_win = jnp.pad(
        table[_NFULL * _WIN :].T, ((0, 0), (0, _WIN - _TAIL))
    )
    scr = _k1(table.T, idx, tail_win)
    out_t = _k2(scr)
    return out_t.T


# speedup vs baseline: 4.2542x; 1.0065x over previous
import functools

import jax
import jax.numpy as jnp
from jax import lax
from jax.experimental import pallas as pl
from jax.experimental.pallas import tpu as pltpu
from jax.experimental.pallas import tpu_sc as plsc

_VOCAB = 1000000
_DIM = 32
_BATCH = 16384

_info = plsc.get_sparse_core_info()
_NC, _NS = _info.num_cores, _info.num_subcores
_NW = _NC * _NS
_BPW = _BATCH // _NW
_WIN = 512
_WSHIFT = 9
_NFULL = _VOCAB // _WIN
_TAILW = _NFULL
_TAIL = _VOCAB - _NFULL * _WIN
_NPAIR = (_NFULL + 2 * _NW - 1) // (2 * _NW)
_YCHUNK = 2048


def _process_window(w, win_ref, gbase, glen, srt_c, srt_r, ext_c, ext_r,
                    rows_stage, ridx_v, flag_sm, scr_hbm, sem_s):
    iota = lax.iota(jnp.int32, 16)

    def scan(i, ns):
        cols = srt_c[pl.ds(gbase + i * 16, 16)]
        rks = srt_r[pl.ds(gbase + i * 16, 16)]
        valid = (i * 16 + iota) < glen
        m = jnp.logical_and((cols >> _WSHIFT) == w, valid)
        plsc.store_compressed(ext_c.at[pl.ds(ns, 16)], cols, mask=m)
        plsc.store_compressed(ext_r.at[pl.ds(ns, 16)], rks, mask=m)
        return ns + plsc.all_reduce_population_count(m)[0]

    nsub = lax.fori_loop(0, (glen + 15) // 16, scan, 0)
    ngrp = (nsub + 15) // 16

    def scatter_copy():
        return pltpu.make_async_copy(
            rows_stage,
            scr_hbm.at[plsc.Indices(ridx_v.at[0], ignored_value=-1)],
            sem_s,
        )

    def drain_scatter():
        @pl.when(flag_sm[0] == 1)
        def _():
            scatter_copy().wait()
            flag_sm[0] = 0

    def gloop(g, _):
        gslot = g & 7

        @pl.when(gslot == 0)
        def _fresh_batch():
            drain_scatter()
            neg1 = jnp.full((16,), -1, jnp.int32)
            for h in range(8):
                ridx_v[0, pl.ds(h * 16, 16)] = neg1

        gb = g * 16
        cols16 = ext_c[pl.ds(gb, 16)] & (_WIN - 1)
        rk16 = ext_r[pl.ds(gb, 16)]
        valid = (gb + iota) < nsub
        pos16 = gslot * 16 + iota
        for d in range(_DIM):
            dfull = jnp.full((16,), d, jnp.int32)
            vals = plsc.load_gather(win_ref, [dfull, cols16])
            plsc.store_scatter(rows_stage, [pos16, dfull], vals)
        ridx_v[0, pl.ds(gslot * 16, 16)] = jnp.where(valid, rk16, -1)

        @pl.when(gslot == 7)
        def _flush():
            scatter_copy().start()
            flag_sm[0] = 1

        return _

    lax.fori_loop(0, ngrp, gloop, 0)

    @pl.when(ngrp % 8 != 0)
    def _flush_tail():
        scatter_copy().start()
        flag_sm[0] = 1


def _make_k1():
    mesh = plsc.VectorSubcoreMesh(core_axis_name="c", subcore_axis_name="s")

    @functools.partial(
        pl.kernel,
        mesh=mesh,
        out_type=jax.ShapeDtypeStruct((_BATCH, 128), jnp.float32),
        scratch_types=[
            pltpu.VMEM((_YCHUNK,), jnp.int32),
            pltpu.VMEM((_BATCH,), jnp.int32),
            pltpu.VMEM((_BATCH,), jnp.int32),
            pltpu.VMEM((2, _DIM, _WIN), jnp.float32),
            pltpu.VMEM((_BATCH,), jnp.int32),
            pltpu.VMEM((_BATCH,), jnp.int32),
            pltpu.VMEM((128, 128), jnp.float32),
            pltpu.VMEM((1, 128), jnp.int32),
            pltpu.SMEM((32,), jnp.int32),
            pltpu.SemaphoreType.DMA,
            pltpu.SemaphoreType.DMA,
            pltpu.SemaphoreType.DMA,
        ],
        compiler_params=pltpu.CompilerParams(needs_layout_passes=False),
    )
    def k1(tb_hbm, idx_hbm, tail_hbm, scr_hbm, ybuf, lcol, lrank, win2,
           sub_c, sub_r, rows_stage, ridx_v, flag_sm, semA, semB, sem_s):
        me = lax.axis_index("s") * _NC + lax.axis_index("c")
        iota = lax.iota(jnp.int32, 16)
        flag_sm[0] = 0

        def stage(w, buf, sem):
            for g in range(_DIM // 8):
                pltpu.make_async_copy(
                    tb_hbm.at[pl.ds(8 * g, 8), pl.ds(w * _WIN, _WIN)],
                    win2.at[buf, pl.ds(8 * g, 8)],
                    sem,
                ).start()

        stage(me, 0, semA)
        stage(me + _NW, 1, semB)

        cnt = 0
        for p in range(_BATCH // _YCHUNK):
            pltpu.sync_copy(idx_hbm.at[pl.ds(p * _YCHUNK, _YCHUNK)], ybuf)

            def build(i, c, _p=p):
                ys = ybuf[pl.ds(i * 16, 16)]
                m = ((ys >> _WSHIFT) & (_NW - 1)) == me
                plsc.store_compressed(lcol.at[pl.ds(c, 16)], ys, mask=m)
                plsc.store_compressed(
                    lrank.at[pl.ds(c, 16)], _p * _YCHUNK + i * 16 + iota,
                    mask=m,
                )
                return c + plsc.all_reduce_population_count(m)[0]

            cnt = lax.fori_loop(0, _YCHUNK // 16, build, cnt)
        nlist = cnt
        nchunk = (nlist + 15) // 16

        for j in range(8):
            flag_sm[8 + j] = 0

        def count_pass(i, _):
            cols = lcol[pl.ds(i * 16, 16)]
            valid = (i * 16 + iota) < nlist
            for j in range(8):
                m = jnp.logical_and((cols >> 17) == j, valid)
                flag_sm[8 + j] = (
                    flag_sm[8 + j] + plsc.all_reduce_population_count(m)[0]
                )
            return _

        lax.fori_loop(0, nchunk, count_pass, 0)
        flag_sm[16] = 0
        flag_sm[24] = 0
        for j in range(1, 8):
            flag_sm[16 + j] = flag_sm[16 + j - 1] + flag_sm[8 + j - 1]
            flag_sm[24 + j] = flag_sm[16 + j]

        def place_pass(i, _):
            cols = lcol[pl.ds(i * 16, 16)]
            rks = lrank[pl.ds(i * 16, 16)]
            valid = (i * 16 + iota) < nlist
            for j in range(8):
                m = jnp.logical_and((cols >> 17) == j, valid)
                o = flag_sm[16 + j]
                plsc.store_compressed(sub_c.at[pl.ds(o, 16)], cols, mask=m)
                plsc.store_compressed(sub_r.at[pl.ds(o, 16)], rks, mask=m)
                flag_sm[16 + j] = o + plsc.all_reduce_population_count(m)[0]
            return _

        lax.fori_loop(0, nchunk, place_pass, 0)

        def stage_wait(buf, sem):
            pltpu.make_async_copy(
                tb_hbm.at[:, pl.ds(0, _WIN)], win2.at[buf], sem
            ).wait()

        def proc(w, buf):
            j = (w - me) >> 8
            _process_window(w, win2.at[buf], flag_sm[24 + j], flag_sm[8 + j],
                            sub_c, sub_r, lcol, lrank, rows_stage, ridx_v,
                            flag_sm, scr_hbm, sem_s)

        def pair(pi, _):
            w0 = me + (2 * pi) * _NW
            w1 = me + (2 * pi + 1) * _NW
            w2 = me + (2 * pi + 2) * _NW
            w3 = me + (2 * pi + 3) * _NW

            @pl.when(w0 < _NFULL)
            def _p0():
                stage_wait(0, semA)
                proc(w0, 0)

            @pl.when(w2 < _NFULL)
            def _s2():
                stage(w2, 0, semA)

            @pl.when(w1 < _NFULL)
            def _p1():
                stage_wait(1, semB)
                proc(w1, 1)

            @pl.when(w3 < _NFULL)
            def _s3():
                stage(w3, 1, semB)

            return _

        lax.fori_loop(0, _NPAIR, pair, 0)

        @pl.when(me == _TAILW % _NW)
        def _tail():
            pltpu.sync_copy(tail_hbm, win2.at[0])
            proc(_TAILW, 0)

        @pl.when(flag_sm[0] == 1)
        def _drain():
            pltpu.make_async_copy(
                rows_stage,
                scr_hbm.at[plsc.Indices(ridx_v.at[0], ignored_value=-1)],
                sem_s,
            ).wait()
            flag_sm[0] = 0

    return k1


def _make_k2():
    mesh = plsc.VectorSubcoreMesh(core_axis_name="c", subcore_axis_name="s")

    @functools.partial(
        pl.kernel,
        mesh=mesh,
        out_type=jax.ShapeDtypeStruct((_DIM, _BATCH), jnp.float32),
        scratch_types=[
            pltpu.VMEM((2, 128, 128), jnp.float32),
            pltpu.VMEM((_DIM, _BPW), jnp.float32),
            pltpu.SemaphoreType.DMA,
            pltpu.SemaphoreType.DMA,
        ],
        compiler_params=pltpu.CompilerParams(needs_layout_passes=False),
    )
    def k2(scr_hbm, out_hbm, rows2_v, cols_v, semA, semB):
        me = lax.axis_index("s") * _NC + lax.axis_index("c")
        iota = lax.iota(jnp.int32, 16)
        base = me * _BPW
        sems = (semA, semB)
        nchunks = _BPW // 128

        def chunk_copy(c, buf):
            return pltpu.make_async_copy(
                scr_hbm.at[pl.ds(base + c * 128, 128)],
                rows2_v.at[buf],
                sems[buf],
            )

        chunk_copy(0, 0).start()
        for c in range(nchunks):
            if c + 1 < nchunks:
                chunk_copy(c + 1, (c + 1) % 2).start()
            chunk_copy(c, c % 2).wait()

            def transpose_d(d, _, _c=c):
                dfull = jnp.full((16,), d, jnp.int32)
                for g in range(8):
                    vals = plsc.load_gather(
                        rows2_v.at[_c % 2], [g * 16 + iota, dfull]
                    )
                    cols_v[d, pl.ds(_c * 128 + g * 16, 16)] = vals
                return _

            lax.fori_loop(0, _DIM, transpose_d, 0)
        pltpu.sync_copy(cols_v, out_hbm.at[:, pl.ds(base, _BPW)])

    return k2


_k1 = _make_k1()
_k2 = _make_k2()


def kernel(y, table):
    idx = y.astype(jnp.int32)
    tail_win = jnp.pad(
        table[_NFULL * _WIN :].T, ((0, 0), (0, _WIN - _TAIL))
    )
    scr = _k1(table.T, idx, tail_win)
    out_t = _k2(scr)
    return out_t.T


# speedup vs baseline: 4.7575x; 1.1183x over previous
import functools

import jax
import jax.numpy as jnp
from jax import lax
from jax.experimental import pallas as pl
from jax.experimental.pallas import tpu as pltpu
from jax.experimental.pallas import tpu_sc as plsc

_VOCAB = 1000000
_DIM = 32
_BATCH = 16384

_info = plsc.get_sparse_core_info()
_NC, _NS = _info.num_cores, _info.num_subcores
_NW = _NC * _NS
_BPW = _BATCH // _NW
_WIN = 512
_WSHIFT = 9
_NFULL = _VOCAB // _WIN
_TAILW = _NFULL
_TAIL = _VOCAB - _NFULL * _WIN
_NPAIR = (_NFULL + 2 * _NW - 1) // (2 * _NW)
_YCHUNK = 2048


def _process_window(w, win_ref, gbase, glen, srt_c, srt_r, ext_c, ext_r,
                    rows_stage, ridx_v, flag_sm, scr_hbm, sem_s):
    iota = lax.iota(jnp.int32, 16)

    def scan(i, ns):
        cols = srt_c[pl.ds(gbase + i * 16, 16)]
        rks = srt_r[pl.ds(gbase + i * 16, 16)]
        valid = (i * 16 + iota) < glen
        m = jnp.logical_and((cols >> _WSHIFT) == w, valid)
        plsc.store_compressed(ext_c.at[pl.ds(ns, 16)], cols, mask=m)
        plsc.store_compressed(ext_r.at[pl.ds(ns, 16)], rks, mask=m)
        return ns + plsc.all_reduce_population_count(m)[0]

    nsub = lax.fori_loop(0, (glen + 15) // 16, scan, 0)
    ngrp = (nsub + 15) // 16

    def scatter_copy():
        return pltpu.make_async_copy(
            rows_stage,
            scr_hbm.at[plsc.Indices(ridx_v.at[0], ignored_value=-1)],
            sem_s,
        )

    def drain_scatter():
        @pl.when(flag_sm[0] == 1)
        def _():
            scatter_copy().wait()
            flag_sm[0] = 0

    def gloop(g, _):
        gslot = g & 7

        @pl.when(gslot == 0)
        def _fresh_batch():
            drain_scatter()
            neg1 = jnp.full((16,), -1, jnp.int32)
            for h in range(8):
                ridx_v[0, pl.ds(h * 16, 16)] = neg1

        gb = g * 16
        cols16 = ext_c[pl.ds(gb, 16)] & (_WIN - 1)
        rk16 = ext_r[pl.ds(gb, 16)]
        valid = (gb + iota) < nsub
        pos16 = gslot * 16 + iota
        for d in range(_DIM):
            dfull = jnp.full((16,), d, jnp.int32)
            vals = plsc.load_gather(win_ref, [dfull, cols16])
            plsc.store_scatter(rows_stage, [pos16, dfull], vals)
        ridx_v[0, pl.ds(gslot * 16, 16)] = jnp.where(valid, rk16, -1)

        @pl.when(gslot == 7)
        def _flush():
            scatter_copy().start()
            flag_sm[0] = 1

        return _

    lax.fori_loop(0, ngrp, gloop, 0)

    @pl.when(ngrp % 8 != 0)
    def _flush_tail():
        scatter_copy().start()
        flag_sm[0] = 1


def _make_k1():
    mesh = plsc.VectorSubcoreMesh(core_axis_name="c", subcore_axis_name="s")

    @functools.partial(
        pl.kernel,
        mesh=mesh,
        out_type=jax.ShapeDtypeStruct((_BATCH, 128), jnp.float32),
        scratch_types=[
            pltpu.VMEM((_YCHUNK,), jnp.int32),
            pltpu.VMEM((_BATCH,), jnp.int32),
            pltpu.VMEM((_BATCH,), jnp.int32),
            pltpu.VMEM((2, _DIM, _WIN), jnp.float32),
            pltpu.VMEM((_BATCH,), jnp.int32),
            pltpu.VMEM((_BATCH,), jnp.int32),
            pltpu.VMEM((128, 128), jnp.float32),
            pltpu.VMEM((1, 128), jnp.int32),
            pltpu.SMEM((32,), jnp.int32),
            pltpu.SemaphoreType.DMA,
            pltpu.SemaphoreType.DMA,
            pltpu.SemaphoreType.DMA,
        ],
        compiler_params=pltpu.CompilerParams(needs_layout_passes=False),
    )
    def k1(tb_hbm, idx_hbm, tail_hbm, scr_hbm, ybuf, lcol, lrank, win2,
           sub_c, sub_r, rows_stage, ridx_v, flag_sm, semA, semB, sem_s):
        me = lax.axis_index("s") * _NC + lax.axis_index("c")
        iota = lax.iota(jnp.int32, 16)
        flag_sm[0] = 0

        def stage(w, buf, sem):
            for g in range(_DIM // 8):
                pltpu.make_async_copy(
                    tb_hbm.at[pl.ds(8 * g, 8), pl.ds(w * _WIN, _WIN)],
                    win2.at[buf, pl.ds(8 * g, 8)],
                    sem,
                ).start()

        stage(me, 0, semA)
        stage(me + _NW, 1, semB)

        cnt = 0
        for p in range(_BATCH // _YCHUNK):
            pltpu.sync_copy(idx_hbm.at[pl.ds(p * _YCHUNK, _YCHUNK)], ybuf)

            def build(i, c, _p=p):
                ys = ybuf[pl.ds(i * 16, 16)]
                m = ((ys >> _WSHIFT) & (_NW - 1)) == me
                plsc.store_compressed(lcol.at[pl.ds(c, 16)], ys, mask=m)
                plsc.store_compressed(
                    lrank.at[pl.ds(c, 16)], _p * _YCHUNK + i * 16 + iota,
                    mask=m,
                )
                return c + plsc.all_reduce_population_count(m)[0]

            cnt = lax.fori_loop(0, _YCHUNK // 16, build, cnt)
        nlist = cnt
        nchunk = (nlist + 15) // 16

        for j in range(8):
            flag_sm[8 + j] = 0

        def count_pass(i, _):
            cols = lcol[pl.ds(i * 16, 16)]
            valid = (i * 16 + iota) < nlist
            for j in range(8):
                m = jnp.logical_and((cols >> 17) == j, valid)
                flag_sm[8 + j] = (
                    flag_sm[8 + j] + plsc.all_reduce_population_count(m)[0]
                )
            return _

        lax.fori_loop(0, nchunk, count_pass, 0)
        flag_sm[16] = 0
        flag_sm[24] = 0
        for j in range(1, 8):
            flag_sm[16 + j] = flag_sm[16 + j - 1] + flag_sm[8 + j - 1]
            flag_sm[24 + j] = flag_sm[16 + j]

        def place_pass(i, _):
            cols = lcol[pl.ds(i * 16, 16)]
            rks = lrank[pl.ds(i * 16, 16)]
            valid = (i * 16 + iota) < nlist
            for j in range(8):
                m = jnp.logical_and((cols >> 17) == j, valid)
                o = flag_sm[16 + j]
                plsc.store_compressed(sub_c.at[pl.ds(o, 16)], cols, mask=m)
                plsc.store_compressed(sub_r.at[pl.ds(o, 16)], rks, mask=m)
                flag_sm[16 + j] = o + plsc.all_reduce_population_count(m)[0]
            return _

        lax.fori_loop(0, nchunk, place_pass, 0)

        def stage_wait(buf, sem):
            pltpu.make_async_copy(
                tb_hbm.at[:, pl.ds(0, _WIN)], win2.at[buf], sem
            ).wait()

        def proc(w, buf):
            j = (w - me) >> 8
            _process_window(w, win2.at[buf], flag_sm[24 + j], flag_sm[8 + j],
                            sub_c, sub_r, lcol, lrank, rows_stage, ridx_v,
                            flag_sm, scr_hbm, sem_s)

        def pair(pi, _):
            w0 = me + (2 * pi) * _NW
            w1 = me + (2 * pi + 1) * _NW
            w2 = me + (2 * pi + 2) * _NW
            w3 = me + (2 * pi + 3) * _NW

            @pl.when(w0 < _NFULL)
            def _p0():
                stage_wait(0, semA)
                proc(w0, 0)

            @pl.when(w2 < _NFULL)
            def _s2():
                stage(w2, 0, semA)

            @pl.when(w1 < _NFULL)
            def _p1():
                stage_wait(1, semB)
                proc(w1, 1)

            @pl.when(w3 < _NFULL)
            def _s3():
                stage(w3, 1, semB)

            return _

        lax.fori_loop(0, _NPAIR, pair, 0)

        @pl.when(me == _TAILW % _NW)
        def _tail():
            pltpu.sync_copy(tail_hbm, win2.at[0])
            proc(_TAILW, 0)

        @pl.when(flag_sm[0] == 1)
        def _drain():
            pltpu.make_async_copy(
                rows_stage,
                scr_hbm.at[plsc.Indices(ridx_v.at[0], ignored_value=-1)],
                sem_s,
            ).wait()
            flag_sm[0] = 0

    return k1


def _make_k2():
    mesh = plsc.VectorSubcoreMesh(core_axis_name="c", subcore_axis_name="s")

    @functools.partial(
        pl.kernel,
        mesh=mesh,
        out_type=jax.ShapeDtypeStruct((_DIM, _BATCH), jnp.float32),
        scratch_types=[
            pltpu.VMEM((2, 128, 128), jnp.float32),
            pltpu.VMEM((_DIM, _BPW), jnp.float32),
            pltpu.SemaphoreType.DMA,
            pltpu.SemaphoreType.DMA,
        ],
        compiler_params=pltpu.CompilerParams(needs_layout_passes=False),
    )
    def k2(scr_hbm, out_hbm, rows2_v, cols_v, semA, semB):
        me = lax.axis_index("s") * _NC + lax.axis_index("c")
        iota = lax.iota(jnp.int32, 16)
        base = me * _BPW
        sems = (semA, semB)
        nchunks = _BPW // 128

        def chunk_copy(c, buf):
            return pltpu.make_async_copy(
                scr_hbm.at[pl.ds(base + c * 128, 128)],
                rows2_v.at[buf],
                sems[buf],
            )

        chunk_copy(0, 0).start()
        for c in range(nchunks):
            if c + 1 < nchunks:
                chunk_copy(c + 1, (c + 1) % 2).start()
            chunk_copy(c, c % 2).wait()

            def transpose_d(d, _, _c=c):
                dfull = jnp.full((16,), d, jnp.int32)
                for g in range(8):
                    vals = plsc.load_gather(
                        rows2_v.at[_c % 2], [g * 16 + iota, dfull]
                    )
                    cols_v[d, pl.ds(_c * 128 + g * 16, 16)] = vals
                return _

            lax.fori_loop(0, _DIM, transpose_d, 0)
        pltpu.sync_copy(cols_v, out_hbm.at[:, pl.ds(base, _BPW)])

    return k2


_k1 = _make_k1()
_k2 = _make_k2()


def kernel(y, table):
    idx = y.astype(jnp.int32)
    tail_win = jnp.pad(
        table[_NFULL * _WIN :].T, ((0, 0), (0, _WIN - _TAIL))
    )
    scr = _k1(table.T, idx, tail_win)
    return scr[:, :_DIM]
